# baseline (speedup 1.0000x reference)
import jax
import jax.numpy as jnp
from jax import lax
from jax.experimental import pallas as pl

EPS = 0.001
NS = 601
SMIN = -300.0

BATCH_PER_BLOCK = 128


def _twohot_block(tv_ref, out_ref):
    x = tv_ref[...]
    tt = jnp.sign(x) * (jnp.sqrt(jnp.abs(x) + 1.0) - 1.0 + EPS * x)
    col = lax.broadcasted_iota(jnp.int32, out_ref.shape, 2)
    sup = col.astype(jnp.float32) + SMIN
    out_ref[...] = jnp.maximum(1.0 - jnp.abs(sup - tt[:, :, None]), 0.0)


def kernel(target_value, supports):
    b, k = target_value.shape
    r = BATCH_PER_BLOCK
    return pl.pallas_call(
        _twohot_block,
        grid=(b // r,),
        in_specs=[pl.BlockSpec((r, k), lambda i: (i, 0))],
        out_specs=pl.BlockSpec((r, k, NS), lambda i: (i, 0, 0)),
        out_shape=jax.ShapeDtypeStruct((b, k, NS), jnp.float32),
    )(target_value)
